# baseline (speedup 1.0000x reference)
import functools

import jax
import jax.numpy as jnp
from jax import lax
from jax.experimental import pallas as pl
from jax.experimental.pallas import tpu as pltpu
from jax.experimental.pallas import tpu_sc as plsc

_BATCH = 4096
_ROT_D = 36
_TRA_D = 18

_INFO = plsc.get_sparse_core_info()
_NW = _INFO.num_cores * _INFO.num_subcores
_BPW = _BATCH // _NW
_RN = _BATCH * _ROT_D
_TN = _BATCH * _TRA_D
_RN_W = _RN // _NW
_TN_W = _TN // _NW

_MESH = plsc.VectorSubcoreMesh(core_axis_name="c", subcore_axis_name="s")


@functools.partial(
    pl.kernel,
    mesh=_MESH,
    out_type=(
        jax.ShapeDtypeStruct((_RN,), jnp.float32),
        jax.ShapeDtypeStruct((_TN,), jnp.float32),
    ),
    scratch_types=[
        pltpu.VMEM((_RN_W,), jnp.int32),
        pltpu.VMEM((_TN_W,), jnp.int32),
        pltpu.VMEM((_RN_W,), jnp.float32),
        pltpu.VMEM((_TN_W,), jnp.float32),
        pltpu.SemaphoreType.DMA,
        pltpu.SemaphoreType.DMA,
    ],
)
def _gather_words(ridx_hbm, tidx_hbm, rot_hbm, tra_hbm, rot_out, tra_out,
                  ridx_v, tidx_v, rot_v, tra_v, sem_r, sem_t):
    wid = lax.axis_index("s") * _INFO.num_cores + lax.axis_index("c")
    rbase = wid * _RN_W
    tbase = wid * _TN_W
    pltpu.sync_copy(ridx_hbm.at[pl.ds(rbase, _RN_W)], ridx_v)
    pltpu.sync_copy(tidx_hbm.at[pl.ds(tbase, _TN_W)], tidx_v)
    cr = pltpu.async_copy(rot_hbm.at[ridx_v], rot_v, sem_r)
    ct = pltpu.async_copy(tra_hbm.at[tidx_v], tra_v, sem_t)
    cr.wait()
    pltpu.sync_copy(rot_v, rot_out.at[pl.ds(rbase, _RN_W)])
    ct.wait()
    pltpu.sync_copy(tra_v, tra_out.at[pl.ds(tbase, _TN_W)])


def kernel(indexes, rotation_table, translation_table):
    n_images, n_domains, _ = rotation_table.shape
    rot_flat = rotation_table.reshape(-1)
    tra_flat = translation_table.reshape(-1)
    ridx = (indexes[:, None] * _ROT_D
            + jnp.arange(_ROT_D, dtype=jnp.int32)[None, :]).reshape(-1)
    tidx = (indexes[:, None] * _TRA_D
            + jnp.arange(_TRA_D, dtype=jnp.int32)[None, :]).reshape(-1)
    rot, tra = _gather_words(ridx, tidx, rot_flat, tra_flat)
    return (
        rot.reshape(_BATCH, n_domains, 6),
        tra.reshape(_BATCH, n_domains, 3),
    )


# speedup vs baseline: 5.1592x; 5.1592x over previous
import functools

import jax
import jax.numpy as jnp
from jax import lax
from jax.experimental import pallas as pl
from jax.experimental.pallas import tpu as pltpu
from jax.experimental.pallas import tpu_sc as plsc

_BATCH = 4096

_INFO = plsc.get_sparse_core_info()
_NW = _INFO.num_cores * _INFO.num_subcores
_BPW = _BATCH // _NW
_CH = 32
_NCH = _BPW // _CH

_MESH = plsc.VectorSubcoreMesh(core_axis_name="c", subcore_axis_name="s")


@functools.partial(
    pl.kernel,
    mesh=_MESH,
    out_type=(
        jax.ShapeDtypeStruct((_BATCH, 6, 6), jnp.float32),
        jax.ShapeDtypeStruct((_BATCH, 6, 3), jnp.float32),
    ),
    scratch_types=[
        pltpu.VMEM((_BPW + 16,), jnp.int32),
        pltpu.VMEM((_CH, 6, 6), jnp.float32),
        pltpu.VMEM((_CH, 6, 3), jnp.float32),
        pltpu.SemaphoreType.DMA,
        pltpu.SemaphoreType.DMA,
    ],
)
def _gather_rows(idx_hbm, rot_hbm, tra_hbm, rot_out, tra_out,
                 idx_v, rot_v, tra_v, sem_r, sem_t):
    wid = lax.axis_index("s") * _INFO.num_cores + lax.axis_index("c")
    base = wid * _BPW
    pltpu.sync_copy(idx_hbm.at[pl.ds(base, _BPW)], idx_v.at[pl.ds(0, _BPW)])

    def chunk(c, _):
        cb = c * _CH

        def fire(i, _):
            idx = idx_v[pl.ds(cb + i, 16)][0]
            pltpu.async_copy(rot_hbm.at[idx], rot_v.at[i], sem_r)
            pltpu.async_copy(tra_hbm.at[idx], tra_v.at[i], sem_t)
            return ()

        lax.fori_loop(0, _CH, fire, ())

        def drain(i, _):
            pltpu.make_async_copy(rot_hbm.at[0], rot_v.at[i], sem_r).wait()
            pltpu.make_async_copy(tra_hbm.at[0], tra_v.at[i], sem_t).wait()
            return ()

        lax.fori_loop(0, _CH, drain, ())
        pltpu.sync_copy(rot_v, rot_out.at[pl.ds(base + cb, _CH)])
        pltpu.sync_copy(tra_v, tra_out.at[pl.ds(base + cb, _CH)])
        return ()

    lax.fori_loop(0, _NCH, chunk, ())


def kernel(indexes, rotation_table, translation_table):
    return _gather_rows(indexes, rotation_table, translation_table)
